# baseline (speedup 1.0000x reference)
import functools

import jax
import jax.numpy as jnp
from jax import lax
from jax.experimental import pallas as pl
from jax.experimental.pallas import tpu as pltpu
from jax.experimental.pallas import tpu_sc as plsc

NC = 2
NS = 16
L = 16
CH = 80
CHH = 128
WB = 40


def _rsqrt16(d):
    i = lax.bitcast_convert_type(d, jnp.int32)
    i = 0x5F3759DF - lax.shift_right_logical(i, 1)
    y = lax.bitcast_convert_type(i, jnp.float32)
    for _ in range(3):
        y = y * (1.5 - 0.5 * d * y * y)
    return y


@functools.lru_cache(maxsize=None)
def _coeff_fn(EPAD, E, N):
    NPT = -(-N // (NS * L)) * L
    NPAD = NPT * NS
    RPT = EPAD // CHH // NS
    RROW = E // CHH
    EPT = EPAD // (NC * NS)
    RPC = EPT // CHH
    PD = 4
    mesh = plsc.VectorSubcoreMesh(core_axis_name="c", subcore_axis_name="s")

    @functools.partial(
        pl.kernel,
        out_type=jax.ShapeDtypeStruct((EPAD,), jnp.float32),
        mesh=mesh,
        compiler_params=pltpu.CompilerParams(needs_layout_passes=False),
        scratch_types=[
            pltpu.VMEM((NPT,), jnp.float32),
            pltpu.VMEM((CHH,), jnp.float32),
            pltpu.VMEM((RPT, CHH), jnp.int32),
            pltpu.VMEM((RPT, CHH), jnp.int32),
            pltpu.VMEM((NPT,), jnp.float32),
            pltpu.VMEM((NPAD,), jnp.float32),
            pltpu.VMEM((NPAD,), jnp.float32),
            pltpu.VMEM((RPC, CHH), jnp.int32),
            pltpu.VMEM((RPC, CHH), jnp.int32),
            pltpu.VMEM((EPT,), jnp.float32),
            pltpu.VMEM((EPT,), jnp.float32),
            pltpu.VMEM_SHARED((NPAD,), jnp.float32),
            pltpu.VMEM_SHARED((NPAD,), jnp.float32),
            pltpu.VMEM_SHARED((NPAD,), jnp.float32),
            pltpu.VMEM_SHARED((NPAD,), jnp.float32),
            pltpu.SemaphoreType.DMA,
        ],
    )
    def coeff(src2_hbm, dst2_hbm, w_hbm, c_hbm,
              hbuf, ones, sidx, didx, nbuf, nrmo_v, nrmi_v,
              sbuf, dbuf, wbuf, cbuf,
              hist_out, hist_in, nrmo_s, nrmi_s, sem):
        cid = lax.axis_index("c")
        tid = lax.axis_index("s")
        nbase = tid * NPT

        zero16 = jnp.zeros((L,), jnp.float32)
        one16 = jnp.ones((L,), jnp.float32)

        @pl.loop(0, NPT // L)
        def _(i):
            hbuf[pl.ds(i * L, L)] = zero16

        @pl.loop(0, CHH // L)
        def _(i):
            ones[pl.ds(i * L, L)] = one16

        pltpu.sync_copy(hbuf, hist_out.at[pl.ds(nbase, NPT)])
        pltpu.sync_copy(hbuf, hist_in.at[pl.ds(nbase, NPT)])

        rb = tid * RPT
        pltpu.sync_copy(src2_hbm.at[pl.ds(rb, RPT)], sidx)
        pltpu.sync_copy(dst2_hbm.at[pl.ds(rb, RPT)], didx)
        plsc.subcore_barrier()

        cnt = jnp.clip(RROW - rb, 0, RPT)

        def h_issue(r):
            pltpu.async_copy(ones, hist_out.at[sidx.at[r]], sem, add=True)
            pltpu.async_copy(ones, hist_in.at[didx.at[r]], sem, add=True)

        def h_wait():
            pltpu.make_async_copy(ones, hist_out.at[sidx.at[0]], sem).wait()

        for r in range(PD):
            h_issue(r)

        @pl.loop(0, cnt - PD)
        def _(j):
            h_wait()
            h_wait()
            h_issue(j + PD)

        for _r in range(2 * PD):
            h_wait()

        plsc.subcore_barrier()

        pltpu.sync_copy(hist_out.at[pl.ds(nbase, NPT)], hbuf)

        @pl.loop(0, NPT // L)
        def _(k):
            deg = hbuf[pl.ds(k * L, L)]
            nbuf[pl.ds(k * L, L)] = _rsqrt16(jnp.maximum(deg, 1.0))

        pltpu.sync_copy(nbuf, nrmo_s.at[pl.ds(nbase, NPT)])
        pltpu.sync_copy(hist_in.at[pl.ds(nbase, NPT)], hbuf)

        @pl.loop(0, NPT // L)
        def _(k):
            deg = hbuf[pl.ds(k * L, L)]
            nbuf[pl.ds(k * L, L)] = _rsqrt16(jnp.maximum(deg, 1.0))

        pltpu.sync_copy(nbuf, nrmi_s.at[pl.ds(nbase, NPT)])
        plsc.subcore_barrier()

        pltpu.sync_copy(nrmo_s, nrmo_v)
        pltpu.sync_copy(nrmi_s, nrmi_v)

        ebase = (cid * NS + tid) * EPT
        erow = (cid * NS + tid) * RPC
        pltpu.sync_copy(src2_hbm.at[pl.ds(erow, RPC)], sbuf)
        pltpu.sync_copy(dst2_hbm.at[pl.ds(erow, RPC)], dbuf)
        pltpu.sync_copy(w_hbm.at[pl.ds(ebase, EPT)], wbuf)

        @pl.loop(0, RPC)
        def _(r):
            for g in range(CHH // L):
                sl = pl.ds(g * L, L)
                ns = plsc.load_gather(nrmo_v, [sbuf[r, sl]])
                nd = plsc.load_gather(nrmi_v, [dbuf[r, sl]])
                off = pl.ds(r * CHH + g * L, L)
                cbuf[off] = wbuf[off] * ns * nd

        pltpu.sync_copy(cbuf, c_hbm.at[pl.ds(ebase, EPT)])

    return coeff


@functools.lru_cache(maxsize=None)
def _prop_fn(EPAD, N, D):
    NPT = -(-N // (NS * L)) * L
    NPAD = NPT * NS
    EPT = EPAD // (NC * NS)
    NCH = EPT // CH
    assert NCH % 4 == 0 and NCH >= 12
    mesh = plsc.VectorSubcoreMesh(core_axis_name="c", subcore_axis_name="s")

    @functools.partial(
        pl.kernel,
        out_type=(jax.ShapeDtypeStruct((N, D), jnp.float32),
                  jax.ShapeDtypeStruct((N, D), jnp.float32)),
        mesh=mesh,
        compiler_params=pltpu.CompilerParams(needs_layout_passes=False),
        scratch_types=[
            pltpu.VMEM((8, CH), jnp.int32),
            pltpu.VMEM((8, CH), jnp.int32),
            pltpu.VMEM((8 * CH,), jnp.float32),
            pltpu.VMEM((CH, D), jnp.float32),
            pltpu.VMEM((CH, D), jnp.float32),
            pltpu.VMEM((CH, D), jnp.float32),
            pltpu.VMEM((CH, D), jnp.float32),
            pltpu.VMEM_SHARED((NPAD, D), jnp.float32),
            pltpu.SemaphoreType.DMA,
            pltpu.SemaphoreType.DMA,
            pltpu.SemaphoreType.DMA,
            pltpu.SemaphoreType.DMA,
            pltpu.SemaphoreType.DMA,
            pltpu.SemaphoreType.DMA,
            pltpu.SemaphoreType.DMA,
            pltpu.SemaphoreType.DMA,
            pltpu.SemaphoreType.DMA,
        ],
    )
    def prop(h_hbm, src_hbm, dst_hbm, c_hbm, out0, out1,
             dring, sring, cring, r0, r1, r2, r3, acc,
             g0, g1, g2, g3, s0, s1, s2, s3, sem_p):
        rows = (r0, r1, r2, r3)
        gsem = (g0, g1, g2, g3)
        ssem = (s0, s1, s2, s3)
        cid = lax.axis_index("c")
        tid = lax.axis_index("s")
        nbase = tid * NPT

        zero16 = jnp.zeros((L,), jnp.float32)

        @pl.loop(0, CH)
        def _(i):
            for jj in range(D // L):
                r0[i, pl.ds(jj * L, L)] = zero16

        for i in range(NPT // CH):
            pltpu.async_copy(r0, acc.at[pl.ds(nbase + i * CH, CH)], g0)
        for i in range(NPT // CH):
            pltpu.make_async_copy(r0, acc.at[pl.ds(nbase, CH)], g0).wait()

        ebase = (cid * NS + tid) * EPT
        plsc.subcore_barrier()

        def p_issue(row):
            slot = lax.rem(row, 8)
            off = pl.ds(ebase + row * CH, CH)
            pltpu.async_copy(src_hbm.at[off], sring.at[slot], sem_p)
            pltpu.async_copy(dst_hbm.at[off], dring.at[slot], sem_p)
            pltpu.async_copy(
                c_hbm.at[off], cring.at[pl.ds(slot * CH, CH)], sem_p)

        def p_wait2():
            for _c in range(3):
                pltpu.make_async_copy(
                    c_hbm.at[pl.ds(ebase, CH)],
                    cring.at[pl.ds(0, CH)], sem_p).wait()

        def g_issue(j, b):
            pltpu.async_copy(
                h_hbm.at[sring.at[lax.rem(j, 8)]], rows[b], gsem[b])

        def g_wait(b):
            pltpu.make_async_copy(
                h_hbm.at[sring.at[0]], rows[b], gsem[b]).wait()

        def s_issue(j, b):
            pltpu.async_copy(
                rows[b], acc.at[dring.at[lax.rem(j, 8)]], ssem[b], add=True)

        def s_wait(b):
            pltpu.make_async_copy(
                rows[b], acc.at[dring.at[0]], ssem[b]).wait()

        def scale(j, b):
            rbuf = rows[b]
            cb = lax.rem(j, 8) * CH

            @pl.loop(0, CH // L)
            def _(k):
                base = cb + k * L
                for m in range(L):
                    cvec = plsc.load_gather(
                        cring, [jnp.full((L,), base + m, jnp.int32)])
                    r = k * L + m
                    for jj in range(D // L):
                        sl = pl.ds(jj * L, L)
                        rbuf[r, sl] = rbuf[r, sl] * cvec

        for r in range(4):
            off = pl.ds(ebase + r * CH, CH)
            pltpu.sync_copy(src_hbm.at[off], sring.at[r])
            pltpu.sync_copy(dst_hbm.at[off], dring.at[r])
            pltpu.sync_copy(c_hbm.at[off], cring.at[pl.ds(r * CH, CH)])
        g_issue(0, 0)
        g_issue(1, 1)
        g_wait(0); scale(0, 0); s_issue(0, 0); g_issue(2, 2); p_issue(4)
        g_wait(1); scale(1, 1); s_issue(1, 1); g_issue(3, 3); p_issue(5)
        p_wait2(); g_wait(2); scale(2, 2); s_issue(2, 2); s_wait(0)
        g_issue(4, 0); p_issue(6)
        p_wait2(); g_wait(3); scale(3, 3); s_issue(3, 3); s_wait(1)
        g_issue(5, 1); p_issue(7)

        @pl.loop(0, (NCH - 8) // 4)
        def _(t):
            for b in range(4):
                j = 4 + 4 * t + b
                p_wait2()
                g_wait(b)
                scale(j, b)
                s_issue(j, b)
                s_wait((b + 2) % 4)
                g_issue(j + 2, (b + 2) % 4)
                p_issue(j + 4)

        p_wait2(); g_wait(0); scale(NCH - 4, 0); s_issue(NCH - 4, 0)
        s_wait(2); g_issue(NCH - 2, 2)
        p_wait2(); g_wait(1); scale(NCH - 3, 1); s_issue(NCH - 3, 1)
        s_wait(3); g_issue(NCH - 1, 3)
        g_wait(2); scale(NCH - 2, 2); s_issue(NCH - 2, 2); s_wait(0)
        g_wait(3); scale(NCH - 1, 3); s_issue(NCH - 1, 3); s_wait(1)
        s_wait(2)
        s_wait(3)
        plsc.subcore_barrier()

        nh = jnp.maximum(jnp.minimum(NPT, N - nbase), 0) // WB

        @pl.loop(0, nh)
        def _(i):
            rbw = nbase + i * WB
            pltpu.sync_copy(acc.at[pl.ds(rbw, WB)], r0.at[pl.ds(0, WB)])

            @pl.when(cid == 0)
            def _():
                pltpu.sync_copy(r0.at[pl.ds(0, WB)], out0.at[pl.ds(rbw, WB)])

            @pl.when(cid == 1)
            def _():
                pltpu.sync_copy(r0.at[pl.ds(0, WB)], out1.at[pl.ds(rbw, WB)])

    return prop


@functools.lru_cache(maxsize=None)
def _combine_fn(N, D, relu):
    BR = 2000
    assert N % BR == 0

    def body(p0_ref, p1_ref, b_ref, o_ref):
        o = p0_ref[...] + p1_ref[...] + b_ref[...]
        if relu:
            o = jnp.maximum(o, 0.0)
        o_ref[...] = o

    return pl.pallas_call(
        body,
        out_shape=jax.ShapeDtypeStruct((N, D), jnp.float32),
        grid=(N // BR,),
        in_specs=[
            pl.BlockSpec((BR, D), lambda i: (i, 0)),
            pl.BlockSpec((BR, D), lambda i: (i, 0)),
            pl.BlockSpec((1, D), lambda i: (0, 0)),
        ],
        out_specs=pl.BlockSpec((BR, D), lambda i: (i, 0)),
    )


def kernel(x, edge_index, edge_weight, bias1, bias2):
    N, D = x.shape
    E = edge_weight.shape[0]
    ALIGN = NC * NS * CH * 8
    EPAD = -(-E // ALIGN) * ALIGN
    pad = EPAD - E
    spread = (jnp.arange(pad, dtype=jnp.int32) * 37) % N
    src1 = jnp.concatenate([edge_index[0], spread])
    dst1 = jnp.concatenate([edge_index[1], spread])
    src2 = src1.reshape(EPAD // CHH, CHH)
    dst2 = dst1.reshape(EPAD // CHH, CHH)
    w = jnp.pad(edge_weight, (0, pad))

    c = _coeff_fn(EPAD, E, N)(src2, dst2, w)
    p0, p1 = _prop_fn(EPAD, N, D)(x, src1, dst1, c)
    h1 = _combine_fn(N, D, True)(p0, p1, bias1.reshape(1, D))
    q0, q1 = _prop_fn(EPAD, N, D)(h1, src1, dst1, c)
    out = _combine_fn(N, D, False)(q0, q1, bias2.reshape(1, D))
    return out


# speedup vs baseline: 1.0985x; 1.0985x over previous
import functools

import jax
import jax.numpy as jnp
from jax import lax
from jax.experimental import pallas as pl
from jax.experimental.pallas import tpu as pltpu
from jax.experimental.pallas import tpu_sc as plsc

NC = 2
NS = 16
L = 16
CH = 80
CHH = 128
WB = 40


def _rsqrt16(d):
    i = lax.bitcast_convert_type(d, jnp.int32)
    i = 0x5F3759DF - lax.shift_right_logical(i, 1)
    y = lax.bitcast_convert_type(i, jnp.float32)
    for _ in range(3):
        y = y * (1.5 - 0.5 * d * y * y)
    return y


@functools.lru_cache(maxsize=None)
def _coeff_fn(EPAD, E, N):
    NPT = -(-N // (NS * L)) * L
    NPAD = NPT * NS
    RPT = EPAD // CHH // NS
    RROW = E // CHH
    EPT = EPAD // (NC * NS)
    RPC = EPT // CHH
    PD = 4
    mesh = plsc.VectorSubcoreMesh(core_axis_name="c", subcore_axis_name="s")

    @functools.partial(
        pl.kernel,
        out_type=jax.ShapeDtypeStruct((EPAD,), jnp.float32),
        mesh=mesh,
        compiler_params=pltpu.CompilerParams(needs_layout_passes=False),
        scratch_types=[
            pltpu.VMEM((NPT,), jnp.float32),
            pltpu.VMEM((CHH,), jnp.float32),
            pltpu.VMEM((RPT, CHH), jnp.int32),
            pltpu.VMEM((RPT, CHH), jnp.int32),
            pltpu.VMEM((NPT,), jnp.float32),
            pltpu.VMEM((NPAD,), jnp.float32),
            pltpu.VMEM((NPAD,), jnp.float32),
            pltpu.VMEM((RPC, CHH), jnp.int32),
            pltpu.VMEM((RPC, CHH), jnp.int32),
            pltpu.VMEM((EPT,), jnp.float32),
            pltpu.VMEM((EPT,), jnp.float32),
            pltpu.VMEM_SHARED((NPAD,), jnp.float32),
            pltpu.VMEM_SHARED((NPAD,), jnp.float32),
            pltpu.VMEM_SHARED((NPAD,), jnp.float32),
            pltpu.VMEM_SHARED((NPAD,), jnp.float32),
            pltpu.SemaphoreType.DMA,
        ],
    )
    def coeff(src2_hbm, dst2_hbm, w_hbm, c_hbm,
              hbuf, ones, sidx, didx, nbuf, nrmo_v, nrmi_v,
              sbuf, dbuf, wbuf, cbuf,
              hist_out, hist_in, nrmo_s, nrmi_s, sem):
        cid = lax.axis_index("c")
        tid = lax.axis_index("s")
        nbase = tid * NPT

        zero16 = jnp.zeros((L,), jnp.float32)
        one16 = jnp.ones((L,), jnp.float32)

        @pl.loop(0, NPT // L)
        def _(i):
            hbuf[pl.ds(i * L, L)] = zero16

        @pl.loop(0, CHH // L)
        def _(i):
            ones[pl.ds(i * L, L)] = one16

        pltpu.sync_copy(hbuf, hist_out.at[pl.ds(nbase, NPT)])
        pltpu.sync_copy(hbuf, hist_in.at[pl.ds(nbase, NPT)])

        rb = tid * RPT
        pltpu.sync_copy(src2_hbm.at[pl.ds(rb, RPT)], sidx)
        pltpu.sync_copy(dst2_hbm.at[pl.ds(rb, RPT)], didx)
        plsc.subcore_barrier()

        cnt = jnp.clip(RROW - rb, 0, RPT)

        def h_issue(r):
            pltpu.async_copy(ones, hist_out.at[sidx.at[r]], sem, add=True)
            pltpu.async_copy(ones, hist_in.at[didx.at[r]], sem, add=True)

        def h_wait():
            pltpu.make_async_copy(ones, hist_out.at[sidx.at[0]], sem).wait()

        for r in range(PD):
            h_issue(r)

        @pl.loop(0, cnt - PD)
        def _(j):
            h_wait()
            h_wait()
            h_issue(j + PD)

        for _r in range(2 * PD):
            h_wait()

        plsc.subcore_barrier()

        pltpu.sync_copy(hist_out.at[pl.ds(nbase, NPT)], hbuf)

        @pl.loop(0, NPT // L)
        def _(k):
            deg = hbuf[pl.ds(k * L, L)]
            nbuf[pl.ds(k * L, L)] = _rsqrt16(jnp.maximum(deg, 1.0))

        pltpu.sync_copy(nbuf, nrmo_s.at[pl.ds(nbase, NPT)])
        pltpu.sync_copy(hist_in.at[pl.ds(nbase, NPT)], hbuf)

        @pl.loop(0, NPT // L)
        def _(k):
            deg = hbuf[pl.ds(k * L, L)]
            nbuf[pl.ds(k * L, L)] = _rsqrt16(jnp.maximum(deg, 1.0))

        pltpu.sync_copy(nbuf, nrmi_s.at[pl.ds(nbase, NPT)])
        plsc.subcore_barrier()

        pltpu.sync_copy(nrmo_s, nrmo_v)
        pltpu.sync_copy(nrmi_s, nrmi_v)

        ebase = (cid * NS + tid) * EPT
        erow = (cid * NS + tid) * RPC
        pltpu.sync_copy(src2_hbm.at[pl.ds(erow, RPC)], sbuf)
        pltpu.sync_copy(dst2_hbm.at[pl.ds(erow, RPC)], dbuf)
        pltpu.sync_copy(w_hbm.at[pl.ds(ebase, EPT)], wbuf)

        @pl.loop(0, RPC)
        def _(r):
            for g in range(CHH // L):
                sl = pl.ds(g * L, L)
                ns = plsc.load_gather(nrmo_v, [sbuf[r, sl]])
                nd = plsc.load_gather(nrmi_v, [dbuf[r, sl]])
                off = pl.ds(r * CHH + g * L, L)
                cbuf[off] = wbuf[off] * ns * nd

        pltpu.sync_copy(cbuf, c_hbm.at[pl.ds(ebase, EPT)])

    return coeff


@functools.lru_cache(maxsize=None)
def _prop_fn(EPAD, N, D):
    NPT = -(-N // (NS * L)) * L
    NPAD = NPT * NS
    EPT = EPAD // (NC * NS)
    NCH = EPT // CH
    assert NCH % 4 == 0 and NCH >= 12
    mesh = plsc.VectorSubcoreMesh(core_axis_name="c", subcore_axis_name="s")

    @functools.partial(
        pl.kernel,
        out_type=(jax.ShapeDtypeStruct((N, D), jnp.float32),
                  jax.ShapeDtypeStruct((N, D), jnp.float32)),
        mesh=mesh,
        compiler_params=pltpu.CompilerParams(needs_layout_passes=False),
        scratch_types=[
            pltpu.VMEM((8, CH), jnp.int32),
            pltpu.VMEM((8, CH), jnp.int32),
            pltpu.VMEM((8 * CH,), jnp.float32),
            pltpu.VMEM((CH, D), jnp.float32),
            pltpu.VMEM((CH, D), jnp.float32),
            pltpu.VMEM((CH, D), jnp.float32),
            pltpu.VMEM((CH, D), jnp.float32),
            pltpu.VMEM_SHARED((NPAD, D), jnp.float32),
            pltpu.SemaphoreType.DMA,
            pltpu.SemaphoreType.DMA,
            pltpu.SemaphoreType.DMA,
            pltpu.SemaphoreType.DMA,
            pltpu.SemaphoreType.DMA,
            pltpu.SemaphoreType.DMA,
            pltpu.SemaphoreType.DMA,
            pltpu.SemaphoreType.DMA,
            pltpu.SemaphoreType.DMA,
        ],
    )
    def prop(h_hbm, src_hbm, dst_hbm, c_hbm, out0, out1,
             dring, sring, cring, r0, r1, r2, r3, acc,
             g0, g1, g2, g3, s0, s1, s2, s3, sem_p):
        rows = (r0, r1, r2, r3)
        gsem = (g0, g1, g2, g3)
        ssem = (s0, s1, s2, s3)
        cid = lax.axis_index("c")
        tid = lax.axis_index("s")
        nbase = tid * NPT

        zero16 = jnp.zeros((L,), jnp.float32)

        @pl.loop(0, CH)
        def _(i):
            for jj in range(D // L):
                r0[i, pl.ds(jj * L, L)] = zero16

        for i in range(NPT // CH):
            pltpu.async_copy(r0, acc.at[pl.ds(nbase + i * CH, CH)], g0)
        for i in range(NPT // CH):
            pltpu.make_async_copy(r0, acc.at[pl.ds(nbase, CH)], g0).wait()

        ebase = (cid * NS + tid) * EPT
        plsc.subcore_barrier()

        def p_issue(row):
            slot = lax.rem(row, 8)
            off = pl.ds(ebase + row * CH, CH)
            pltpu.async_copy(src_hbm.at[off], sring.at[slot], sem_p)
            pltpu.async_copy(dst_hbm.at[off], dring.at[slot], sem_p)
            pltpu.async_copy(
                c_hbm.at[off], cring.at[pl.ds(slot * CH, CH)], sem_p)

        def p_wait2():
            for _c in range(3):
                pltpu.make_async_copy(
                    c_hbm.at[pl.ds(ebase, CH)],
                    cring.at[pl.ds(0, CH)], sem_p).wait()

        def g_issue(j, b):
            pltpu.async_copy(
                h_hbm.at[sring.at[lax.rem(j, 8)]], rows[b], gsem[b])

        def g_wait(b):
            pltpu.make_async_copy(
                h_hbm.at[sring.at[0]], rows[b], gsem[b]).wait()

        def s_issue(j, b):
            pltpu.async_copy(
                rows[b], acc.at[dring.at[lax.rem(j, 8)]], ssem[b], add=True)

        def s_wait(b):
            pltpu.make_async_copy(
                rows[b], acc.at[dring.at[0]], ssem[b]).wait()

        def scale(j, b):
            rbuf = rows[b]
            cb = lax.rem(j, 8) * CH

            @pl.loop(0, CH // L)
            def _(k):
                cv = cring[pl.ds(cb + k * L, L)]
                for m in range(L):
                    ci = cv[m]
                    r = k * L + m
                    for jj in range(D // L):
                        sl = pl.ds(jj * L, L)
                        rbuf[r, sl] = rbuf[r, sl] * ci

        for r in range(4):
            off = pl.ds(ebase + r * CH, CH)
            pltpu.sync_copy(src_hbm.at[off], sring.at[r])
            pltpu.sync_copy(dst_hbm.at[off], dring.at[r])
            pltpu.sync_copy(c_hbm.at[off], cring.at[pl.ds(r * CH, CH)])
        g_issue(0, 0)
        g_issue(1, 1)
        g_wait(0); scale(0, 0); s_issue(0, 0); g_issue(2, 2); p_issue(4)
        g_wait(1); scale(1, 1); s_issue(1, 1); g_issue(3, 3); p_issue(5)
        p_wait2(); g_wait(2); scale(2, 2); s_issue(2, 2); s_wait(0)
        g_issue(4, 0); p_issue(6)
        p_wait2(); g_wait(3); scale(3, 3); s_issue(3, 3); s_wait(1)
        g_issue(5, 1); p_issue(7)

        @pl.loop(0, (NCH - 8) // 4)
        def _(t):
            for b in range(4):
                j = 4 + 4 * t + b
                p_wait2()
                g_wait(b)
                scale(j, b)
                s_issue(j, b)
                s_wait((b + 2) % 4)
                g_issue(j + 2, (b + 2) % 4)
                p_issue(j + 4)

        p_wait2(); g_wait(0); scale(NCH - 4, 0); s_issue(NCH - 4, 0)
        s_wait(2); g_issue(NCH - 2, 2)
        p_wait2(); g_wait(1); scale(NCH - 3, 1); s_issue(NCH - 3, 1)
        s_wait(3); g_issue(NCH - 1, 3)
        g_wait(2); scale(NCH - 2, 2); s_issue(NCH - 2, 2); s_wait(0)
        g_wait(3); scale(NCH - 1, 3); s_issue(NCH - 1, 3); s_wait(1)
        s_wait(2)
        s_wait(3)
        plsc.subcore_barrier()

        nh = jnp.maximum(jnp.minimum(NPT, N - nbase), 0) // WB

        @pl.loop(0, nh)
        def _(i):
            rbw = nbase + i * WB
            pltpu.sync_copy(acc.at[pl.ds(rbw, WB)], r0.at[pl.ds(0, WB)])

            @pl.when(cid == 0)
            def _():
                pltpu.sync_copy(r0.at[pl.ds(0, WB)], out0.at[pl.ds(rbw, WB)])

            @pl.when(cid == 1)
            def _():
                pltpu.sync_copy(r0.at[pl.ds(0, WB)], out1.at[pl.ds(rbw, WB)])

    return prop


@functools.lru_cache(maxsize=None)
def _combine_fn(N, D, relu):
    BR = 2000
    assert N % BR == 0

    def body(p0_ref, p1_ref, b_ref, o_ref):
        o = p0_ref[...] + p1_ref[...] + b_ref[...]
        if relu:
            o = jnp.maximum(o, 0.0)
        o_ref[...] = o

    return pl.pallas_call(
        body,
        out_shape=jax.ShapeDtypeStruct((N, D), jnp.float32),
        grid=(N // BR,),
        in_specs=[
            pl.BlockSpec((BR, D), lambda i: (i, 0)),
            pl.BlockSpec((BR, D), lambda i: (i, 0)),
            pl.BlockSpec((1, D), lambda i: (0, 0)),
        ],
        out_specs=pl.BlockSpec((BR, D), lambda i: (i, 0)),
    )


def kernel(x, edge_index, edge_weight, bias1, bias2):
    N, D = x.shape
    E = edge_weight.shape[0]
    ALIGN = NC * NS * CH * 8
    EPAD = -(-E // ALIGN) * ALIGN
    pad = EPAD - E
    spread = (jnp.arange(pad, dtype=jnp.int32) * 37) % N
    src1 = jnp.concatenate([edge_index[0], spread])
    dst1 = jnp.concatenate([edge_index[1], spread])
    src2 = src1.reshape(EPAD // CHH, CHH)
    dst2 = dst1.reshape(EPAD // CHH, CHH)
    w = jnp.pad(edge_weight, (0, pad))

    c = _coeff_fn(EPAD, E, N)(src2, dst2, w)
    p0, p1 = _prop_fn(EPAD, N, D)(x, src1, dst1, c)
    h1 = _combine_fn(N, D, True)(p0, p1, bias1.reshape(1, D))
    q0, q1 = _prop_fn(EPAD, N, D)(h1, src1, dst1, c)
    out = _combine_fn(N, D, False)(q0, q1, bias2.reshape(1, D))
    return out


# speedup vs baseline: 1.1139x; 1.0141x over previous
import functools

import jax
import jax.numpy as jnp
from jax import lax
from jax.experimental import pallas as pl
from jax.experimental.pallas import tpu as pltpu
from jax.experimental.pallas import tpu_sc as plsc

NC = 2
NS = 16
L = 16
CH = 80
CHH = 128
WB = 40


def _rsqrt16(d):
    i = lax.bitcast_convert_type(d, jnp.int32)
    i = 0x5F3759DF - lax.shift_right_logical(i, 1)
    y = lax.bitcast_convert_type(i, jnp.float32)
    for _ in range(3):
        y = y * (1.5 - 0.5 * d * y * y)
    return y


@functools.lru_cache(maxsize=None)
def _coeff_fn(EPAD, E, N):
    NPT = -(-N // (NS * L)) * L
    NPAD = NPT * NS
    RPT = EPAD // CHH // NS
    RROW = E // CHH
    EPT = EPAD // (NC * NS)
    RPC = EPT // CHH
    PD = 4
    mesh = plsc.VectorSubcoreMesh(core_axis_name="c", subcore_axis_name="s")

    @functools.partial(
        pl.kernel,
        out_type=jax.ShapeDtypeStruct((EPAD,), jnp.float32),
        mesh=mesh,
        compiler_params=pltpu.CompilerParams(needs_layout_passes=False),
        scratch_types=[
            pltpu.VMEM((NPT,), jnp.float32),
            pltpu.VMEM((CHH,), jnp.float32),
            pltpu.VMEM((RPT, CHH), jnp.int32),
            pltpu.VMEM((RPT, CHH), jnp.int32),
            pltpu.VMEM((NPT,), jnp.float32),
            pltpu.VMEM((NPAD,), jnp.float32),
            pltpu.VMEM((NPAD,), jnp.float32),
            pltpu.VMEM((RPC, CHH), jnp.int32),
            pltpu.VMEM((RPC, CHH), jnp.int32),
            pltpu.VMEM((EPT,), jnp.float32),
            pltpu.VMEM((EPT,), jnp.float32),
            pltpu.VMEM_SHARED((NPAD,), jnp.float32),
            pltpu.VMEM_SHARED((NPAD,), jnp.float32),
            pltpu.VMEM_SHARED((NPAD,), jnp.float32),
            pltpu.VMEM_SHARED((NPAD,), jnp.float32),
            pltpu.SemaphoreType.DMA,
        ],
    )
    def coeff(src2_hbm, dst2_hbm, w_hbm, c_hbm,
              hbuf, ones, sidx, didx, nbuf, nrmo_v, nrmi_v,
              sbuf, dbuf, wbuf, cbuf,
              hist_out, hist_in, nrmo_s, nrmi_s, sem):
        cid = lax.axis_index("c")
        tid = lax.axis_index("s")
        nbase = tid * NPT

        zero16 = jnp.zeros((L,), jnp.float32)
        one16 = jnp.ones((L,), jnp.float32)

        @pl.loop(0, NPT // L)
        def _(i):
            hbuf[pl.ds(i * L, L)] = zero16

        @pl.loop(0, CHH // L)
        def _(i):
            ones[pl.ds(i * L, L)] = one16

        pltpu.sync_copy(hbuf, hist_out.at[pl.ds(nbase, NPT)])
        pltpu.sync_copy(hbuf, hist_in.at[pl.ds(nbase, NPT)])

        rb = tid * RPT
        pltpu.sync_copy(src2_hbm.at[pl.ds(rb, RPT)], sidx)
        pltpu.sync_copy(dst2_hbm.at[pl.ds(rb, RPT)], didx)
        plsc.subcore_barrier()

        cnt = jnp.clip(RROW - rb, 0, RPT)

        def h_issue(r):
            pltpu.async_copy(ones, hist_out.at[sidx.at[r]], sem, add=True)
            pltpu.async_copy(ones, hist_in.at[didx.at[r]], sem, add=True)

        def h_wait():
            pltpu.make_async_copy(ones, hist_out.at[sidx.at[0]], sem).wait()

        for r in range(PD):
            h_issue(r)

        @pl.loop(0, cnt - PD)
        def _(j):
            h_wait()
            h_wait()
            h_issue(j + PD)

        for _r in range(2 * PD):
            h_wait()

        plsc.subcore_barrier()

        pltpu.sync_copy(hist_out.at[pl.ds(nbase, NPT)], hbuf)

        @pl.loop(0, NPT // L)
        def _(k):
            deg = hbuf[pl.ds(k * L, L)]
            nbuf[pl.ds(k * L, L)] = _rsqrt16(jnp.maximum(deg, 1.0))

        pltpu.sync_copy(nbuf, nrmo_s.at[pl.ds(nbase, NPT)])
        pltpu.sync_copy(hist_in.at[pl.ds(nbase, NPT)], hbuf)

        @pl.loop(0, NPT // L)
        def _(k):
            deg = hbuf[pl.ds(k * L, L)]
            nbuf[pl.ds(k * L, L)] = _rsqrt16(jnp.maximum(deg, 1.0))

        pltpu.sync_copy(nbuf, nrmi_s.at[pl.ds(nbase, NPT)])
        plsc.subcore_barrier()

        pltpu.sync_copy(nrmo_s, nrmo_v)
        pltpu.sync_copy(nrmi_s, nrmi_v)

        ebase = (cid * NS + tid) * EPT
        erow = (cid * NS + tid) * RPC
        pltpu.sync_copy(src2_hbm.at[pl.ds(erow, RPC)], sbuf)
        pltpu.sync_copy(dst2_hbm.at[pl.ds(erow, RPC)], dbuf)
        pltpu.sync_copy(w_hbm.at[pl.ds(ebase, EPT)], wbuf)

        @pl.loop(0, RPC)
        def _(r):
            for g in range(CHH // L):
                sl = pl.ds(g * L, L)
                ns = plsc.load_gather(nrmo_v, [sbuf[r, sl]])
                nd = plsc.load_gather(nrmi_v, [dbuf[r, sl]])
                off = pl.ds(r * CHH + g * L, L)
                cbuf[off] = wbuf[off] * ns * nd

        pltpu.sync_copy(cbuf, c_hbm.at[pl.ds(ebase, EPT)])

    return coeff


@functools.lru_cache(maxsize=None)
def _prop_fn(EPAD, N, D):
    NPT = -(-N // (NS * L)) * L
    NPAD = NPT * NS
    EPT = EPAD // (NC * NS)
    NCH = EPT // CH
    assert NCH % 4 == 0 and NCH >= 12
    mesh = plsc.VectorSubcoreMesh(core_axis_name="c", subcore_axis_name="s")

    @functools.partial(
        pl.kernel,
        out_type=(jax.ShapeDtypeStruct((N, D), jnp.float32),
                  jax.ShapeDtypeStruct((N, D), jnp.float32)),
        mesh=mesh,
        compiler_params=pltpu.CompilerParams(needs_layout_passes=False),
        scratch_types=[
            pltpu.VMEM((8, CH), jnp.int32),
            pltpu.VMEM((8, CH), jnp.int32),
            pltpu.VMEM((8 * CH,), jnp.float32),
            pltpu.VMEM((CH, D), jnp.float32),
            pltpu.VMEM((CH, D), jnp.float32),
            pltpu.VMEM((CH, D), jnp.float32),
            pltpu.VMEM((CH, D), jnp.float32),
            pltpu.VMEM_SHARED((NPAD, D), jnp.float32),
            pltpu.SemaphoreType.DMA,
            pltpu.SemaphoreType.DMA,
            pltpu.SemaphoreType.DMA,
            pltpu.SemaphoreType.DMA,
            pltpu.SemaphoreType.DMA,
            pltpu.SemaphoreType.DMA,
            pltpu.SemaphoreType.DMA,
            pltpu.SemaphoreType.DMA,
            pltpu.SemaphoreType.DMA,
        ],
    )
    def prop(h_hbm, src_hbm, dst_hbm, c_hbm, out0, out1,
             dring, sring, cring, r0, r1, r2, r3, acc,
             g0, g1, g2, g3, s0, s1, s2, s3, sem_p):
        rows = (r0, r1, r2, r3)
        gsem = (g0, g1, g2, g3)
        ssem = (s0, s1, s2, s3)
        cid = lax.axis_index("c")
        tid = lax.axis_index("s")
        nbase = tid * NPT

        zero16 = jnp.zeros((L,), jnp.float32)

        @pl.loop(0, CH)
        def _(i):
            for jj in range(D // L):
                r0[i, pl.ds(jj * L, L)] = zero16

        for i in range(NPT // CH):
            pltpu.async_copy(r0, acc.at[pl.ds(nbase + i * CH, CH)], g0)
        for i in range(NPT // CH):
            pltpu.make_async_copy(r0, acc.at[pl.ds(nbase, CH)], g0).wait()

        ebase = (cid * NS + tid) * EPT
        plsc.subcore_barrier()

        def p_issue(row):
            slot = lax.rem(row, 8)
            off = pl.ds(ebase + row * CH, CH)
            pltpu.async_copy(src_hbm.at[off], sring.at[slot], sem_p)
            pltpu.async_copy(dst_hbm.at[off], dring.at[slot], sem_p)
            pltpu.async_copy(
                c_hbm.at[off], cring.at[pl.ds(slot * CH, CH)], sem_p)

        def p_wait2():
            for _c in range(3):
                pltpu.make_async_copy(
                    c_hbm.at[pl.ds(ebase, CH)],
                    cring.at[pl.ds(0, CH)], sem_p).wait()

        def g_issue(j, b):
            pltpu.async_copy(
                h_hbm.at[sring.at[lax.rem(j, 8)]], rows[b], gsem[b])

        def g_wait(b):
            pltpu.make_async_copy(
                h_hbm.at[sring.at[0]], rows[b], gsem[b]).wait()

        def s_issue(j, b):
            pltpu.async_copy(
                rows[b], acc.at[dring.at[lax.rem(j, 8)]], ssem[b], add=True)

        def s_wait(b):
            pltpu.make_async_copy(
                rows[b], acc.at[dring.at[0]], ssem[b]).wait()

        def scale(j, b):
            rbuf = rows[b]
            cb = lax.rem(j, 8) * CH

            @pl.loop(0, CH // L)
            def _(k):
                cv = cring[pl.ds(cb + k * L, L)]
                for m in range(L):
                    ci = cv[m]
                    r = k * L + m
                    for jj in range(D // L):
                        sl = pl.ds(jj * L, L)
                        rbuf[r, sl] = rbuf[r, sl] * ci

        for r in range(4):
            off = pl.ds(ebase + r * CH, CH)
            pltpu.sync_copy(src_hbm.at[off], sring.at[r])
            pltpu.sync_copy(dst_hbm.at[off], dring.at[r])
            pltpu.sync_copy(c_hbm.at[off], cring.at[pl.ds(r * CH, CH)])
        g_issue(0, 0)
        g_issue(1, 1)
        g_wait(0); scale(0, 0); s_issue(0, 0); g_issue(2, 2); p_issue(4)
        g_wait(1); scale(1, 1); s_issue(1, 1); g_issue(3, 3); p_issue(5)
        p_wait2(); g_wait(2); scale(2, 2); s_issue(2, 2); s_wait(0)
        g_issue(4, 0); p_issue(6)
        p_wait2(); g_wait(3); scale(3, 3); s_issue(3, 3); s_wait(1)
        g_issue(5, 1); p_issue(7)

        @pl.loop(0, (NCH - 8) // 4)
        def _(t):
            for b in range(4):
                j = 4 + 4 * t + b
                p_wait2()
                g_wait(b)
                s_wait((b + 2) % 4)
                g_issue(j + 2, (b + 2) % 4)
                p_issue(j + 4)
                scale(j, b)
                s_issue(j, b)

        p_wait2(); g_wait(0); scale(NCH - 4, 0); s_issue(NCH - 4, 0)
        s_wait(2); g_issue(NCH - 2, 2)
        p_wait2(); g_wait(1); scale(NCH - 3, 1); s_issue(NCH - 3, 1)
        s_wait(3); g_issue(NCH - 1, 3)
        g_wait(2); scale(NCH - 2, 2); s_issue(NCH - 2, 2); s_wait(0)
        g_wait(3); scale(NCH - 1, 3); s_issue(NCH - 1, 3); s_wait(1)
        s_wait(2)
        s_wait(3)
        plsc.subcore_barrier()

        nh = jnp.maximum(jnp.minimum(NPT, N - nbase), 0) // WB

        @pl.loop(0, nh)
        def _(i):
            rbw = nbase + i * WB
            pltpu.sync_copy(acc.at[pl.ds(rbw, WB)], r0.at[pl.ds(0, WB)])

            @pl.when(cid == 0)
            def _():
                pltpu.sync_copy(r0.at[pl.ds(0, WB)], out0.at[pl.ds(rbw, WB)])

            @pl.when(cid == 1)
            def _():
                pltpu.sync_copy(r0.at[pl.ds(0, WB)], out1.at[pl.ds(rbw, WB)])

    return prop


@functools.lru_cache(maxsize=None)
def _combine_fn(N, D, relu):
    BR = 2000
    assert N % BR == 0

    def body(p0_ref, p1_ref, b_ref, o_ref):
        o = p0_ref[...] + p1_ref[...] + b_ref[...]
        if relu:
            o = jnp.maximum(o, 0.0)
        o_ref[...] = o

    return pl.pallas_call(
        body,
        out_shape=jax.ShapeDtypeStruct((N, D), jnp.float32),
        grid=(N // BR,),
        in_specs=[
            pl.BlockSpec((BR, D), lambda i: (i, 0)),
            pl.BlockSpec((BR, D), lambda i: (i, 0)),
            pl.BlockSpec((1, D), lambda i: (0, 0)),
        ],
        out_specs=pl.BlockSpec((BR, D), lambda i: (i, 0)),
    )


def kernel(x, edge_index, edge_weight, bias1, bias2):
    N, D = x.shape
    E = edge_weight.shape[0]
    ALIGN = NC * NS * CH * 8
    EPAD = -(-E // ALIGN) * ALIGN
    pad = EPAD - E
    spread = (jnp.arange(pad, dtype=jnp.int32) * 37) % N
    src1 = jnp.concatenate([edge_index[0], spread])
    dst1 = jnp.concatenate([edge_index[1], spread])
    src2 = src1.reshape(EPAD // CHH, CHH)
    dst2 = dst1.reshape(EPAD // CHH, CHH)
    w = jnp.pad(edge_weight, (0, pad))

    c = _coeff_fn(EPAD, E, N)(src2, dst2, w)
    p0, p1 = _prop_fn(EPAD, N, D)(x, src1, dst1, c)
    h1 = _combine_fn(N, D, True)(p0, p1, bias1.reshape(1, D))
    q0, q1 = _prop_fn(EPAD, N, D)(h1, src1, dst1, c)
    out = _combine_fn(N, D, False)(q0, q1, bias2.reshape(1, D))
    return out


# speedup vs baseline: 1.1406x; 1.0240x over previous
import functools

import jax
import jax.numpy as jnp
from jax import lax
from jax.experimental import pallas as pl
from jax.experimental.pallas import tpu as pltpu
from jax.experimental.pallas import tpu_sc as plsc

NC = 2
NS = 16
L = 16
CH = 80
CHH = 128
WB = 40


def _rsqrt16(d):
    i = lax.bitcast_convert_type(d, jnp.int32)
    i = 0x5F3759DF - lax.shift_right_logical(i, 1)
    y = lax.bitcast_convert_type(i, jnp.float32)
    for _ in range(3):
        y = y * (1.5 - 0.5 * d * y * y)
    return y


@functools.lru_cache(maxsize=None)
def _coeff_fn(EPAD, E, N):
    NPT = -(-N // (NS * L)) * L
    NPAD = NPT * NS
    RPT = EPAD // CHH // NS
    RROW = E // CHH
    EPT = EPAD // (NC * NS)
    RPC = EPT // CHH
    PD = 8
    mesh = plsc.VectorSubcoreMesh(core_axis_name="c", subcore_axis_name="s")

    @functools.partial(
        pl.kernel,
        out_type=jax.ShapeDtypeStruct((EPAD,), jnp.float32),
        mesh=mesh,
        compiler_params=pltpu.CompilerParams(needs_layout_passes=False),
        scratch_types=[
            pltpu.VMEM((NPT,), jnp.float32),
            pltpu.VMEM((CHH,), jnp.float32),
            pltpu.VMEM((RPT, CHH), jnp.int32),
            pltpu.VMEM((RPT, CHH), jnp.int32),
            pltpu.VMEM((NPT,), jnp.float32),
            pltpu.VMEM((NPAD,), jnp.float32),
            pltpu.VMEM((NPAD,), jnp.float32),
            pltpu.VMEM((RPC, CHH), jnp.int32),
            pltpu.VMEM((RPC, CHH), jnp.int32),
            pltpu.VMEM((EPT,), jnp.float32),
            pltpu.VMEM((EPT,), jnp.float32),
            pltpu.VMEM_SHARED((NPAD,), jnp.float32),
            pltpu.VMEM_SHARED((NPAD,), jnp.float32),
            pltpu.VMEM_SHARED((NPAD,), jnp.float32),
            pltpu.VMEM_SHARED((NPAD,), jnp.float32),
            pltpu.SemaphoreType.DMA,
        ],
    )
    def coeff(src2_hbm, dst2_hbm, w_hbm, c_hbm,
              hbuf, ones, sidx, didx, nbuf, nrmo_v, nrmi_v,
              sbuf, dbuf, wbuf, cbuf,
              hist_out, hist_in, nrmo_s, nrmi_s, sem):
        cid = lax.axis_index("c")
        tid = lax.axis_index("s")
        nbase = tid * NPT

        zero16 = jnp.zeros((L,), jnp.float32)
        one16 = jnp.ones((L,), jnp.float32)

        @pl.loop(0, NPT // L)
        def _(i):
            hbuf[pl.ds(i * L, L)] = zero16

        @pl.loop(0, CHH // L)
        def _(i):
            ones[pl.ds(i * L, L)] = one16

        pltpu.sync_copy(hbuf, hist_out.at[pl.ds(nbase, NPT)])
        pltpu.sync_copy(hbuf, hist_in.at[pl.ds(nbase, NPT)])

        rb = tid * RPT
        pltpu.sync_copy(src2_hbm.at[pl.ds(rb, RPT)], sidx)
        pltpu.sync_copy(dst2_hbm.at[pl.ds(rb, RPT)], didx)
        plsc.subcore_barrier()

        cnt = jnp.clip(RROW - rb, 0, RPT)

        def h_issue(r):
            pltpu.async_copy(ones, hist_out.at[sidx.at[r]], sem, add=True)
            pltpu.async_copy(ones, hist_in.at[didx.at[r]], sem, add=True)

        def h_wait():
            pltpu.make_async_copy(ones, hist_out.at[sidx.at[0]], sem).wait()

        for r in range(PD):
            h_issue(r)

        @pl.loop(0, cnt - PD)
        def _(j):
            h_wait()
            h_wait()
            h_issue(j + PD)

        for _r in range(2 * PD):
            h_wait()

        plsc.subcore_barrier()

        pltpu.sync_copy(hist_out.at[pl.ds(nbase, NPT)], hbuf)

        @pl.loop(0, NPT // L)
        def _(k):
            deg = hbuf[pl.ds(k * L, L)]
            nbuf[pl.ds(k * L, L)] = _rsqrt16(jnp.maximum(deg, 1.0))

        pltpu.sync_copy(nbuf, nrmo_s.at[pl.ds(nbase, NPT)])
        pltpu.sync_copy(hist_in.at[pl.ds(nbase, NPT)], hbuf)

        @pl.loop(0, NPT // L)
        def _(k):
            deg = hbuf[pl.ds(k * L, L)]
            nbuf[pl.ds(k * L, L)] = _rsqrt16(jnp.maximum(deg, 1.0))

        pltpu.sync_copy(nbuf, nrmi_s.at[pl.ds(nbase, NPT)])
        plsc.subcore_barrier()

        pltpu.sync_copy(nrmo_s, nrmo_v)
        pltpu.sync_copy(nrmi_s, nrmi_v)

        ebase = (cid * NS + tid) * EPT
        erow = (cid * NS + tid) * RPC
        pltpu.sync_copy(src2_hbm.at[pl.ds(erow, RPC)], sbuf)
        pltpu.sync_copy(dst2_hbm.at[pl.ds(erow, RPC)], dbuf)
        pltpu.sync_copy(w_hbm.at[pl.ds(ebase, EPT)], wbuf)

        @pl.loop(0, RPC)
        def _(r):
            for g in range(CHH // L):
                sl = pl.ds(g * L, L)
                ns = plsc.load_gather(nrmo_v, [sbuf[r, sl]])
                nd = plsc.load_gather(nrmi_v, [dbuf[r, sl]])
                off = pl.ds(r * CHH + g * L, L)
                cbuf[off] = wbuf[off] * ns * nd

        pltpu.sync_copy(cbuf, c_hbm.at[pl.ds(ebase, EPT)])

    return coeff


@functools.lru_cache(maxsize=None)
def _prop_fn(EPAD, N, D):
    NPT = -(-N // (NS * L)) * L
    NPAD = NPT * NS
    EPT = EPAD // (NC * NS)
    NCH = EPT // CH
    assert NCH % 4 == 0 and NCH >= 12
    mesh = plsc.VectorSubcoreMesh(core_axis_name="c", subcore_axis_name="s")

    @functools.partial(
        pl.kernel,
        out_type=(jax.ShapeDtypeStruct((N, D), jnp.float32),
                  jax.ShapeDtypeStruct((N, D), jnp.float32)),
        mesh=mesh,
        compiler_params=pltpu.CompilerParams(needs_layout_passes=False),
        scratch_types=[
            pltpu.VMEM((8, CH), jnp.int32),
            pltpu.VMEM((8, CH), jnp.int32),
            pltpu.VMEM((8 * CH,), jnp.float32),
            pltpu.VMEM((CH, D), jnp.float32),
            pltpu.VMEM((CH, D), jnp.float32),
            pltpu.VMEM((CH, D), jnp.float32),
            pltpu.VMEM((CH, D), jnp.float32),
            pltpu.VMEM_SHARED((NPAD, D), jnp.float32),
            pltpu.SemaphoreType.DMA,
            pltpu.SemaphoreType.DMA,
            pltpu.SemaphoreType.DMA,
            pltpu.SemaphoreType.DMA,
            pltpu.SemaphoreType.DMA,
            pltpu.SemaphoreType.DMA,
            pltpu.SemaphoreType.DMA,
            pltpu.SemaphoreType.DMA,
            pltpu.SemaphoreType.DMA,
        ],
    )
    def prop(h_hbm, src_hbm, dst_hbm, c_hbm, out0, out1,
             dring, sring, cring, r0, r1, r2, r3, acc,
             g0, g1, g2, g3, s0, s1, s2, s3, sem_p):
        rows = (r0, r1, r2, r3)
        gsem = (g0, g1, g2, g3)
        ssem = (s0, s1, s2, s3)
        cid = lax.axis_index("c")
        tid = lax.axis_index("s")
        nbase = tid * NPT

        zero16 = jnp.zeros((L,), jnp.float32)

        @pl.loop(0, CH)
        def _(i):
            for jj in range(D // L):
                r0[i, pl.ds(jj * L, L)] = zero16

        for i in range(NPT // CH):
            pltpu.async_copy(r0, acc.at[pl.ds(nbase + i * CH, CH)], g0)
        for i in range(NPT // CH):
            pltpu.make_async_copy(r0, acc.at[pl.ds(nbase, CH)], g0).wait()

        ebase = (cid * NS + tid) * EPT
        plsc.subcore_barrier()

        def p_issue(row):
            slot = lax.rem(row, 8)
            off = pl.ds(ebase + row * CH, CH)
            pltpu.async_copy(src_hbm.at[off], sring.at[slot], sem_p)
            pltpu.async_copy(dst_hbm.at[off], dring.at[slot], sem_p)
            pltpu.async_copy(
                c_hbm.at[off], cring.at[pl.ds(slot * CH, CH)], sem_p)

        def p_wait2():
            for _c in range(3):
                pltpu.make_async_copy(
                    c_hbm.at[pl.ds(ebase, CH)],
                    cring.at[pl.ds(0, CH)], sem_p).wait()

        def g_issue(j, b):
            pltpu.async_copy(
                h_hbm.at[sring.at[lax.rem(j, 8)]], rows[b], gsem[b])

        def g_wait(b):
            pltpu.make_async_copy(
                h_hbm.at[sring.at[0]], rows[b], gsem[b]).wait()

        def s_issue(j, b):
            pltpu.async_copy(
                rows[b], acc.at[dring.at[lax.rem(j, 8)]], ssem[b], add=True)

        def s_wait(b):
            pltpu.make_async_copy(
                rows[b], acc.at[dring.at[0]], ssem[b]).wait()

        def scale(j, b):
            rbuf = rows[b]
            cb = lax.rem(j, 8) * CH

            @pl.loop(0, CH // L)
            def _(k):
                cv = cring[pl.ds(cb + k * L, L)]
                for m in range(L):
                    ci = cv[m]
                    r = k * L + m
                    for jj in range(D // L):
                        sl = pl.ds(jj * L, L)
                        rbuf[r, sl] = rbuf[r, sl] * ci

        for r in range(4):
            off = pl.ds(ebase + r * CH, CH)
            pltpu.sync_copy(src_hbm.at[off], sring.at[r])
            pltpu.sync_copy(dst_hbm.at[off], dring.at[r])
            pltpu.sync_copy(c_hbm.at[off], cring.at[pl.ds(r * CH, CH)])
        g_issue(0, 0)
        g_issue(1, 1)
        g_wait(0); scale(0, 0); s_issue(0, 0); g_issue(2, 2); p_issue(4)
        g_wait(1); scale(1, 1); s_issue(1, 1); g_issue(3, 3); p_issue(5)
        p_wait2(); g_wait(2); scale(2, 2); s_issue(2, 2); s_wait(0)
        g_issue(4, 0); p_issue(6)
        p_wait2(); g_wait(3); scale(3, 3); s_issue(3, 3); s_wait(1)
        g_issue(5, 1); p_issue(7)

        @pl.loop(0, (NCH - 8) // 4)
        def _(t):
            for b in range(4):
                j = 4 + 4 * t + b
                p_wait2()
                g_wait(b)
                s_wait((b + 2) % 4)
                g_issue(j + 2, (b + 2) % 4)
                p_issue(j + 4)
                scale(j, b)
                s_issue(j, b)

        p_wait2(); g_wait(0); scale(NCH - 4, 0); s_issue(NCH - 4, 0)
        s_wait(2); g_issue(NCH - 2, 2)
        p_wait2(); g_wait(1); scale(NCH - 3, 1); s_issue(NCH - 3, 1)
        s_wait(3); g_issue(NCH - 1, 3)
        g_wait(2); scale(NCH - 2, 2); s_issue(NCH - 2, 2); s_wait(0)
        g_wait(3); scale(NCH - 1, 3); s_issue(NCH - 1, 3); s_wait(1)
        s_wait(2)
        s_wait(3)
        plsc.subcore_barrier()

        nh2 = jnp.maximum(jnp.minimum(NPT, N - nbase), 0) // WB // 2
        wbuf2 = (r0.at[pl.ds(0, WB)], r1.at[pl.ds(0, WB)])

        @pl.loop(0, nh2)
        def _(i):
            for b2 in range(2):
                idx = 2 * i + b2
                rbw = nbase + idx * WB

                @pl.when(i > 0)
                def _():
                    pltpu.make_async_copy(
                        wbuf2[b2], out0.at[pl.ds(nbase, WB)], gsem[b2]).wait()

                pltpu.sync_copy(acc.at[pl.ds(rbw, WB)], wbuf2[b2])

                @pl.when(cid == 0)
                def _():
                    pltpu.async_copy(
                        wbuf2[b2], out0.at[pl.ds(rbw, WB)], gsem[b2])

                @pl.when(cid == 1)
                def _():
                    pltpu.async_copy(
                        wbuf2[b2], out1.at[pl.ds(rbw, WB)], gsem[b2])

        for b2 in range(2):
            pltpu.make_async_copy(
                wbuf2[b2], out0.at[pl.ds(nbase, WB)], gsem[b2]).wait()

    return prop


@functools.lru_cache(maxsize=None)
def _combine_fn(N, D, relu):
    BR = 2000
    assert N % BR == 0

    def body(p0_ref, p1_ref, b_ref, o_ref):
        o = p0_ref[...] + p1_ref[...] + b_ref[...]
        if relu:
            o = jnp.maximum(o, 0.0)
        o_ref[...] = o

    return pl.pallas_call(
        body,
        out_shape=jax.ShapeDtypeStruct((N, D), jnp.float32),
        grid=(N // BR,),
        in_specs=[
            pl.BlockSpec((BR, D), lambda i: (i, 0)),
            pl.BlockSpec((BR, D), lambda i: (i, 0)),
            pl.BlockSpec((1, D), lambda i: (0, 0)),
        ],
        out_specs=pl.BlockSpec((BR, D), lambda i: (i, 0)),
    )


def kernel(x, edge_index, edge_weight, bias1, bias2):
    N, D = x.shape
    E = edge_weight.shape[0]
    ALIGN = NC * NS * CH * 8
    EPAD = -(-E // ALIGN) * ALIGN
    pad = EPAD - E
    spread = (jnp.arange(pad, dtype=jnp.int32) * 37) % N
    src1 = jnp.concatenate([edge_index[0], spread])
    dst1 = jnp.concatenate([edge_index[1], spread])
    src2 = src1.reshape(EPAD // CHH, CHH)
    dst2 = dst1.reshape(EPAD // CHH, CHH)
    w = jnp.pad(edge_weight, (0, pad))

    c = _coeff_fn(EPAD, E, N)(src2, dst2, w)
    p0, p1 = _prop_fn(EPAD, N, D)(x, src1, dst1, c)
    h1 = _combine_fn(N, D, True)(p0, p1, bias1.reshape(1, D))
    q0, q1 = _prop_fn(EPAD, N, D)(h1, src1, dst1, c)
    out = _combine_fn(N, D, False)(q0, q1, bias2.reshape(1, D))
    return out
